# baseline (device time: 36126 ns/iter reference)
import jax
import jax.numpy as jnp
from jax import lax
from jax.experimental import pallas as pl
from jax.experimental.pallas import tpu as pltpu


def kernel(ids, E):
    v_per, d = E.shape
    t_len = ids.shape[0]
    shift = (v_per - 1).bit_length()

    z = lax.axis_index("z")
    local = ids.astype(jnp.int32) - z * v_per
    mask = (local >= 0) & (local < v_per)
    tok = jnp.arange(t_len, dtype=jnp.int32)
    sentinel = jnp.int32(1 << 30)
    packed = jnp.where(
        mask,
        (tok << shift) | jnp.clip(local, 0, v_per - 1),
        sentinel + tok,
    )
    cum_own = jnp.cumsum(mask.astype(jnp.int32))
    n_own = cum_own[-1]
    pos = jnp.where(mask, cum_own - 1, n_own + tok - cum_own)
    packed = jnp.zeros((t_len,), jnp.int32).at[pos].set(
        packed, unique_indices=True
    )

    def body(pk_ref, e_ref, out_ref, local_sem, send_sem, recv_sem):
        my_x = lax.axis_index("x")
        my_y = lax.axis_index("y")
        my_z = lax.axis_index("z")
        partner = (my_x, my_y, 1 - my_z)

        barrier = pltpu.get_barrier_semaphore()
        pl.semaphore_signal(
            barrier, inc=1, device_id=partner,
            device_id_type=pl.DeviceIdType.MESH,
        )
        pl.semaphore_wait(barrier, 1)

        sent = jnp.int32(1 << 30)
        n_mine = jnp.int32(0)
        step = t_len
        while step >= 1:
            cand = n_mine + step
            probe = pk_ref[jnp.minimum(cand, t_len) - 1]
            ok = jnp.logical_and(cand <= t_len, probe < sent)
            n_mine = jnp.where(ok, cand, n_mine)
            step //= 2
        n_peer = t_len - n_mine

        def issue_rdma(i, c):
            v = pk_ref[i]
            pltpu.make_async_remote_copy(
                src_ref=e_ref.at[pl.ds(v & (v_per - 1), 1), :],
                dst_ref=out_ref.at[pl.ds(v >> shift, 1), :],
                send_sem=send_sem,
                recv_sem=recv_sem,
                device_id=partner,
                device_id_type=pl.DeviceIdType.MESH,
            ).start()
            return c

        def issue_local(i, c):
            v = pk_ref[i]
            pltpu.make_async_copy(
                e_ref.at[pl.ds(v & (v_per - 1), 1), :],
                out_ref.at[pl.ds(v >> shift, 1), :],
                local_sem,
            ).start()
            return c

        lax.fori_loop(0, n_mine, issue_rdma, 0)
        lax.fori_loop(0, n_mine, issue_local, 0)

        def dummy_rdma(k):
            return pltpu.make_async_remote_copy(
                src_ref=e_ref.at[pl.ds(0, k), :],
                dst_ref=out_ref.at[pl.ds(0, k), :],
                send_sem=send_sem,
                recv_sem=recv_sem,
                device_id=partner,
                device_id_type=pl.DeviceIdType.MESH,
            )

        for k in (1024, 512, 256, 128, 64, 32, 16, 8, 4, 2, 1):

            @pl.when((n_peer & k) != 0)
            def _(k=k):
                dummy_rdma(k).wait_recv()

            @pl.when((n_mine & k) != 0)
            def _(k=k):
                pltpu.make_async_copy(
                    e_ref.at[pl.ds(0, k), :],
                    out_ref.at[pl.ds(0, k), :],
                    local_sem,
                ).wait()
                dummy_rdma(k).wait_send()

    return pl.pallas_call(
        body,
        out_shape=jax.ShapeDtypeStruct((t_len, d), jnp.float32),
        in_specs=[
            pl.BlockSpec(memory_space=pltpu.SMEM),
            pl.BlockSpec(memory_space=pl.ANY),
        ],
        out_specs=pl.BlockSpec(memory_space=pltpu.VMEM),
        scratch_shapes=[
            pltpu.SemaphoreType.DMA,
            pltpu.SemaphoreType.DMA,
            pltpu.SemaphoreType.DMA,
        ],
        compiler_params=pltpu.CompilerParams(collective_id=11),
    )(packed, E)
